# Initial kernel scaffold; baseline (speedup 1.0000x reference)
import jax
import jax.numpy as jnp
from jax.experimental import pallas as pl


def kernel(src, dst, adjacent, NodeEmbedding, ncn_W1, ncn_b1, ncn_g, ncn_beta, ncn_W2, ncn_b2, ncn_W3, ncn_b3, ncn_W4, ncn_b4, out_W1, out_b1, out_g, out_beta, out_W2, out_b2, out_W3, out_b3, out_W4, out_b4):
    raise NotImplementedError("write your pallas kernel here")


# speedup vs baseline: 1.1829x; 1.1829x over previous
import functools

import jax
import jax.numpy as jnp
from jax.experimental import pallas as pl
from jax.experimental.pallas import tpu as pltpu

N = 1024
D = 64
B = 8
IN_F = 2 * D
HID = 2 * IN_F
NPAIR = 2 * B


def _mlp_rows(x, W1, b1, g, beta, W2, b2, W3, b3, W4r, b4):
    h = jnp.maximum(jnp.dot(x, W1, preferred_element_type=jnp.float32) + b1, 0.0)
    mu = jnp.mean(h, axis=-1, keepdims=True)
    var = jnp.mean((h - mu) ** 2, axis=-1, keepdims=True)
    h = (h - mu) * jax.lax.rsqrt(var + 1e-5) * g + beta
    h = jnp.maximum(jnp.dot(h, W2, preferred_element_type=jnp.float32) + b2, 0.0)
    h = jnp.maximum(jnp.dot(h, W3, preferred_element_type=jnp.float32) + b3, 0.0)
    return jnp.sum(h * W4r, axis=-1, keepdims=True) + b4


def _body(nodes_ref, adj_ref, E_ref,
          nW1, nb1, ng, nbeta, nW2, nb2, nW3, nb3, nW4r, nb4,
          oW1, ob1, og, obeta, oW2, ob2, oW3, ob3, oW4r, ob4,
          out_ref, cn_ref):
    (nW1, nb1, ng, nbeta, nW2, nb2, nW3, nb3, nW4r, nb4,
     oW1, ob1, og, obeta, oW2, ob2, oW3, ob3, oW4r, ob4) = (
        r[...] for r in (nW1, nb1, ng, nbeta, nW2, nb2, nW3, nb3, nW4r, nb4,
                         oW1, ob1, og, obeta, oW2, ob2, oW3, ob3, oW4r, ob4))
    E = E_ref[...]
    nodes = nodes_ref[0:1, :]
    row_ids = jax.lax.broadcasted_iota(jnp.int32, (N, NPAIR), 0)
    onehot = (row_ids == nodes).astype(jnp.float32)

    nbcols = jnp.dot(adj_ref[...], onehot, preferred_element_type=jnp.float32)
    erows = jax.lax.dot_general(onehot, E, (((0,), (0,)), ((), ())),
                                preferred_element_type=jnp.float32)

    KB = 256
    for c in range(NPAIR * D // KB):
        me_chunk = jnp.concatenate(
            [nbcols[:, p:p + 1] * E for p in range(c * 4, c * 4 + 4)], axis=1)
        acc = jnp.zeros((N, KB), jnp.float32)
        for kb in range(N // KB):
            acc += jnp.dot(adj_ref[:, kb * KB:(kb + 1) * KB],
                           me_chunk[kb * KB:(kb + 1) * KB, :],
                           preferred_element_type=jnp.float32)
        cn_ref[:, c * KB:(c + 1) * KB] = acc

    nb_tar = nbcols[:, 0:B]
    nb_src = nbcols[:, B:NPAIR]

    contribs = []
    for p in range(NPAIR):
        e_other = erows[p:p + 1, :]
        x = jnp.concatenate([E * e_other,
                             cn_ref[:, p * D:(p + 1) * D]], axis=1)
        logit = _mlp_rows(x, nW1, nb1, ng, nbeta, nW2, nb2, nW3, nb3, nW4r, nb4)
        a = jax.nn.sigmoid(logit)
        b = p % B
        if p < B:
            w = nb_src[:, b:b + 1] * (1.0 - nb_tar[:, b:b + 1])
        else:
            w = (1.0 - nb_src[:, b:b + 1]) * nb_tar[:, b:b + 1]
        contribs.append(jax.lax.dot_general(w * a, E, (((0,), (0,)), ((), ())),
                                            preferred_element_type=jnp.float32))
    contrib = jnp.concatenate(contribs, axis=0)

    both = nb_src * nb_tar
    both_e = jax.lax.dot_general(both, E, (((0,), (0,)), ((), ())),
                                 preferred_element_type=jnp.float32)
    all_cn = both_e + contrib[0:B, :] + contrib[B:NPAIR, :]
    prod = erows[B:NPAIR, :] * erows[0:B, :]
    final = jnp.concatenate([prod, all_cn], axis=1)
    out_ref[...] = _mlp_rows(final, oW1, ob1, og, obeta, oW2, ob2, oW3, ob3,
                             oW4r, ob4)


@jax.jit
def _run(nodes, adjf, E, *weights):
    full = lambda a: pl.BlockSpec(a.shape, lambda: (0,) * a.ndim)
    args = (nodes, adjf, E) + weights
    return pl.pallas_call(
        _body,
        out_shape=jax.ShapeDtypeStruct((B, 1), jnp.float32),
        in_specs=[full(a) for a in args],
        out_specs=pl.BlockSpec((B, 1), lambda: (0, 0)),
        scratch_shapes=[pltpu.VMEM((N, NPAIR * D), jnp.float32)],
    )(*args)


def kernel(src, dst, adjacent, NodeEmbedding,
           ncn_W1, ncn_b1, ncn_g, ncn_beta, ncn_W2, ncn_b2, ncn_W3, ncn_b3,
           ncn_W4, ncn_b4,
           out_W1, out_b1, out_g, out_beta, out_W2, out_b2, out_W3, out_b3,
           out_W4, out_b4):
    nodes = jnp.broadcast_to(
        jnp.concatenate([dst, src]).reshape(1, NPAIR), (8, NPAIR))
    adjf = adjacent.astype(jnp.float32)
    r2 = lambda v: v.reshape(1, -1)
    weights = (
        ncn_W1, r2(ncn_b1), r2(ncn_g), r2(ncn_beta), ncn_W2, r2(ncn_b2),
        ncn_W3, r2(ncn_b3), ncn_W4.reshape(1, HID), r2(ncn_b4),
        out_W1, r2(out_b1), r2(out_g), r2(out_beta), out_W2, r2(out_b2),
        out_W3, r2(out_b3), out_W4.reshape(1, HID), r2(out_b4),
    )
    return _run(nodes, adjf, NodeEmbedding, *weights)
